# baseline (speedup 1.0000x reference)
import jax
import jax.numpy as jnp
from jax.experimental import pallas as pl


def _mask_cast_kernel(m_ref, o_ref):
    o_ref[...] = m_ref[...] != 0.0


def _mask_to_bool(mask):
    return pl.pallas_call(
        _mask_cast_kernel,
        out_shape=jax.ShapeDtypeStruct(mask.shape, jnp.bool_),
    )(mask)


def kernel(x, v, mask=None, uu=None):
    if mask is None:
        mask = jnp.ones_like(x[:, :1])
    return (x, v, _mask_to_bool(mask), uu)


# speedup vs baseline: 1.0112x; 1.0112x over previous
import jax
import jax.numpy as jnp
from jax.experimental import pallas as pl


def _trim_kernel(x_ref, v_ref, m_ref, xo_ref, vo_ref, mo_ref):
    xo_ref[...] = x_ref[...]
    vo_ref[...] = v_ref[...]
    mo_ref[...] = m_ref[...] != 0.0


def _trim(x, v, mask):
    B = x.shape[0]
    return pl.pallas_call(
        _trim_kernel,
        grid=(B,),
        in_specs=[
            pl.BlockSpec((1,) + x.shape[1:], lambda b: (b, 0, 0)),
            pl.BlockSpec((1,) + v.shape[1:], lambda b: (b, 0, 0)),
            pl.BlockSpec((1,) + mask.shape[1:], lambda b: (b, 0, 0)),
        ],
        out_specs=[
            pl.BlockSpec((1,) + x.shape[1:], lambda b: (b, 0, 0)),
            pl.BlockSpec((1,) + v.shape[1:], lambda b: (b, 0, 0)),
            pl.BlockSpec((1,) + mask.shape[1:], lambda b: (b, 0, 0)),
        ],
        out_shape=[
            jax.ShapeDtypeStruct(x.shape, x.dtype),
            jax.ShapeDtypeStruct(v.shape, v.dtype),
            jax.ShapeDtypeStruct(mask.shape, jnp.bool_),
        ],
    )(x, v, mask)


def kernel(x, v, mask=None, uu=None):
    if mask is None:
        mask = jnp.ones_like(x[:, :1])
    xo, vo, mo = _trim(x, v, mask)
    return (xo, vo, mo, uu)
